# Initial kernel scaffold; baseline (speedup 1.0000x reference)
import jax
import jax.numpy as jnp
from jax.experimental import pallas as pl


def kernel(queries, keys, values, attn_mask):
    raise NotImplementedError("write your pallas kernel here")


# speedup vs baseline: 4.9098x; 4.9098x over previous
import functools
import math

import jax
import jax.numpy as jnp
import numpy as np
from jax.experimental import pallas as pl
from jax.experimental.pallas import tpu as pltpu

_L = 2048
_D = 64
_H = 16
_U = 40
_KB = 512
_CB = 256
_SCALE = 1.0 / math.sqrt(_D)


def _counts_t_np() -> np.ndarray:
    idx = np.asarray(jax.random.randint(jax.random.key(42), (_L, _U), 0, _L))
    cnt = np.zeros((_L, _L), np.float32)
    np.add.at(cnt, (idx.ravel(), np.repeat(np.arange(_L), _U)), 1.0)
    return cnt


_CNT_T = _counts_t_np()


def _head_kernel(q_ref, k_ref, v_ref, cnt_ref, o_ref, idx_ref, qr_ref, upd_ref):
    q = q_ref[0]
    v = v_ref[0]

    lane = jax.lax.broadcasted_iota(jnp.int32, (1, _L), 1)
    mrun = jnp.full((1, _L), -jnp.inf, jnp.float32)
    srun = jnp.zeros((1, _L), jnp.float32)
    for b in range(_L // _KB):
        kb = k_ref[0, pl.ds(b * _KB, _KB), :]
        st = jax.lax.dot_general(kb, q, (((1,), (1,)), ((), ())),
                                 preferred_element_type=jnp.float32)
        ct = cnt_ref[pl.ds(b * _KB, _KB), :]
        biased = jnp.where(ct > 0.0, st, -jnp.inf)
        mrun = jnp.maximum(mrun, jnp.max(biased, axis=0, keepdims=True))
        srun = srun + jnp.sum(ct * st, axis=0, keepdims=True)
    m = mrun - srun * (1.0 / _L)

    def topk_body(u, mv):
        mx = jnp.max(mv)
        pos = jnp.min(jnp.where(mv == mx, lane, _L))
        idx_ref[u] = pos
        return jnp.where(lane == pos, -jnp.inf, mv)

    jax.lax.fori_loop(0, _U, topk_body, m)

    def gather_body(u, carry):
        i = idx_ref[u]
        qr_ref[pl.ds(u, 1), :] = q_ref[0, pl.ds(i, 1), :]
        return carry

    jax.lax.fori_loop(0, _U, gather_body, 0)

    qr = qr_ref[...]
    sc = jax.lax.dot_general(qr, k_ref[0], (((1,), (1,)), ((), ())),
                             preferred_element_type=jnp.float32) * _SCALE
    sc = sc - jnp.max(sc, axis=1, keepdims=True)
    e = jnp.exp(sc)
    attn = e / jnp.sum(e, axis=1, keepdims=True)
    upd_ref[...] = jnp.dot(attn, v, preferred_element_type=jnp.float32)

    r_io = jax.lax.broadcasted_iota(jnp.int32, (_CB, _CB), 0)
    c_io = jax.lax.broadcasted_iota(jnp.int32, (_CB, _CB), 1)
    tri = (r_io >= c_io).astype(jnp.float32)
    row1 = jax.lax.broadcasted_iota(jnp.int32, (_CB, 1), 0).astype(jnp.float32)
    carry = jnp.zeros((1, _D), jnp.float32)
    for i in range(_L // _CB):
        vb = v[i * _CB:(i + 1) * _CB, :]
        cs = jnp.dot(tri, vb, preferred_element_type=jnp.float32, precision=jax.lax.Precision.HIGHEST) + carry
        o_ref[0, pl.ds(i * _CB, _CB), :] = cs / (row1 + (i * _CB + 1.0))
        carry = carry + jnp.sum(vb, axis=0, keepdims=True)

    def scat_body(u, carry):
        i = idx_ref[u]
        o_ref[0, pl.ds(i, 1), :] = upd_ref[pl.ds(u, 1), :]
        return carry

    jax.lax.fori_loop(0, _U, scat_body, 0)


@jax.jit
def _run(qh, kh, vh):
    cnt_t = jnp.asarray(_CNT_T)
    return pl.pallas_call(
        _head_kernel,
        grid=(_H,),
        in_specs=[
            pl.BlockSpec((1, _L, _D), lambda h: (h, 0, 0)),
            pl.BlockSpec((1, _L, _D), lambda h: (h, 0, 0)),
            pl.BlockSpec((1, _L, _D), lambda h: (h, 0, 0)),
            pl.BlockSpec((_L, _L), lambda h: (0, 0)),
        ],
        out_specs=pl.BlockSpec((1, _L, _D), lambda h: (h, 0, 0)),
        out_shape=jax.ShapeDtypeStruct((_H, _L, _D), jnp.float32),
        scratch_shapes=[
            pltpu.SMEM((_U,), jnp.int32),
            pltpu.VMEM((_U, _D), jnp.float32),
            pltpu.VMEM((_U, _D), jnp.float32),
        ],
    )(qh, kh, vh, cnt_t)


def kernel(queries, keys, values, attn_mask):
    qh = jnp.transpose(queries[0], (1, 0, 2))
    kh = jnp.transpose(keys[0], (1, 0, 2))
    vh = jnp.transpose(values[0], (1, 0, 2))
    ctx = _run(qh, kh, vh)
    return jnp.transpose(ctx, (1, 0, 2))[None]


# speedup vs baseline: 8.9733x; 1.8276x over previous
import functools
import math

import jax
import jax.numpy as jnp
import numpy as np
from jax.experimental import pallas as pl
from jax.experimental.pallas import tpu as pltpu

_L = 2048
_D = 64
_H = 16
_U = 40
_KB = 512
_CB = 256
_SCALE = 1.0 / math.sqrt(_D)
_HIGH = jax.lax.Precision.HIGHEST


def _tf_rounds(x0, x1, rots):
    for r in rots:
        x0 = (x0 + x1).astype(np.uint32)
        x1 = ((x1 << np.uint32(r)) | (x1 >> np.uint32(32 - r))).astype(np.uint32)
        x1 = x0 ^ x1
    return x0, x1


def _threefry2x32(k1, k2, x1, x2):
    ks0, ks1 = np.uint32(k1), np.uint32(k2)
    ks2 = np.uint32(ks0 ^ ks1 ^ np.uint32(0x1BD11BDA))
    r0, r1 = (13, 15, 26, 6), (17, 29, 16, 24)
    x0 = (x1 + ks0).astype(np.uint32)
    y1 = (x2 + ks1).astype(np.uint32)
    for rots, ka, kb, i in ((r0, ks1, ks2, 1), (r1, ks2, ks0, 2),
                            (r0, ks0, ks1, 3), (r1, ks1, ks2, 4),
                            (r0, ks2, ks0, 5)):
        x0, y1 = _tf_rounds(x0, y1, rots)
        x0 = (x0 + ka).astype(np.uint32)
        y1 = (y1 + kb + np.uint32(i)).astype(np.uint32)
    return x0, y1


def _sample_index_np() -> np.ndarray:
    b1, b2 = _threefry2x32(np.uint32(0), np.uint32(42),
                           np.zeros(2, np.uint32), np.arange(2, dtype=np.uint32))
    n = _L * _U
    o1, o2 = _threefry2x32(np.uint32(b1[1]), np.uint32(b2[1]),
                           np.zeros(n, np.uint32), np.arange(n, dtype=np.uint32))
    bits = o1 ^ o2
    return (bits % np.uint32(_L)).astype(np.int32).reshape(_L, _U)


def _counts_t_np() -> np.ndarray:
    idx = _sample_index_np()
    cnt = np.zeros((_L, _L), np.float32)
    np.add.at(cnt, (idx.ravel(), np.repeat(np.arange(_L), _U)), 1.0)
    return cnt


_CNT_T = _counts_t_np()


def _stats_kernel(q_ref, k_ref, v_ref, cnt_ref, m_ref, o_ref):
    q = q_ref[0]
    v = v_ref[0]

    mrun = jnp.full((1, _L), -jnp.inf, jnp.float32)
    srun = jnp.zeros((1, _L), jnp.float32)
    for b in range(_L // _KB):
        kb = k_ref[0, pl.ds(b * _KB, _KB), :]
        st = jax.lax.dot_general(kb, q, (((1,), (1,)), ((), ())),
                                 preferred_element_type=jnp.float32)
        ct = cnt_ref[pl.ds(b * _KB, _KB), :]
        biased = jnp.where(ct > 0.0, st, -jnp.inf)
        mrun = jnp.maximum(mrun, jnp.max(biased, axis=0, keepdims=True))
        srun = srun + jnp.sum(ct * st, axis=0, keepdims=True)
    m_ref[0] = mrun - srun * (1.0 / _L)

    r_io = jax.lax.broadcasted_iota(jnp.int32, (_CB, _CB), 0)
    c_io = jax.lax.broadcasted_iota(jnp.int32, (_CB, _CB), 1)
    tri = (r_io >= c_io).astype(jnp.float32)
    row1 = jax.lax.broadcasted_iota(jnp.int32, (_CB, 1), 0).astype(jnp.float32)
    carry = jnp.zeros((1, _D), jnp.float32)
    for i in range(_L // _CB):
        vb = v[i * _CB:(i + 1) * _CB, :]
        cs = jnp.dot(tri, vb, preferred_element_type=jnp.float32,
                     precision=_HIGH) + carry
        o_ref[0, pl.ds(i * _CB, _CB), :] = cs / (row1 + (i * _CB + 1.0))
        carry = carry + jnp.sum(vb, axis=0, keepdims=True)


def _topk_kernel(m_ref, idx_ref):
    mv = m_ref[:, 0, :]
    lane = jax.lax.broadcasted_iota(jnp.int32, (_H, _L), 1)
    for u in range(_U):
        mx = jnp.max(mv, axis=1, keepdims=True)
        pos = jnp.min(jnp.where(mv == mx, lane, _L), axis=1, keepdims=True)
        idx_ref[:, u:u + 1] = pos
        mv = jnp.where(lane == pos, -jnp.inf, mv)


def _attn_kernel(idx_ref, q_ref, k_ref, v_ref, c_ref, o_ref):
    q = q_ref[0]
    k = k_ref[0]
    v = v_ref[0]
    idxrow = idx_ref[0]

    sub = jax.lax.broadcasted_iota(jnp.int32, (_L, 1), 0)
    oht = (sub == idxrow).astype(jnp.float32)

    qr = jax.lax.dot_general(oht, q, (((0,), (0,)), ((), ())),
                             preferred_element_type=jnp.float32,
                             precision=_HIGH)

    sc = jax.lax.dot_general(qr, k, (((1,), (1,)), ((), ())),
                             preferred_element_type=jnp.float32) * _SCALE
    sc = sc - jnp.max(sc, axis=1, keepdims=True)
    e = jnp.exp(sc)
    attn = e / jnp.sum(e, axis=1, keepdims=True)
    upd = jnp.dot(attn, v, preferred_element_type=jnp.float32)

    scattered = jnp.dot(oht, upd, preferred_element_type=jnp.float32,
                        precision=_HIGH)
    selrow = jnp.sum(oht, axis=1, keepdims=True)
    o_ref[0] = jnp.where(selrow > 0.5, scattered, c_ref[0])


@jax.jit
def _run(qh, kh, vh):
    cnt_t = jnp.asarray(_CNT_T)
    m_all, ctx0 = pl.pallas_call(
        _stats_kernel,
        grid=(_H,),
        in_specs=[
            pl.BlockSpec((1, _L, _D), lambda h: (h, 0, 0)),
            pl.BlockSpec((1, _L, _D), lambda h: (h, 0, 0)),
            pl.BlockSpec((1, _L, _D), lambda h: (h, 0, 0)),
            pl.BlockSpec((_L, _L), lambda h: (0, 0)),
        ],
        out_specs=[
            pl.BlockSpec((1, 1, _L), lambda h: (h, 0, 0)),
            pl.BlockSpec((1, _L, _D), lambda h: (h, 0, 0)),
        ],
        out_shape=[
            jax.ShapeDtypeStruct((_H, 1, _L), jnp.float32),
            jax.ShapeDtypeStruct((_H, _L, _D), jnp.float32),
        ],
    )(qh, kh, vh, cnt_t)

    idx = pl.pallas_call(
        _topk_kernel,
        out_shape=jax.ShapeDtypeStruct((_H, _U), jnp.int32),
    )(m_all)

    idx3 = idx.reshape(_H, 1, _U)
    ctx = pl.pallas_call(
        _attn_kernel,
        grid=(_H,),
        in_specs=[
            pl.BlockSpec((1, 1, _U), lambda h: (h, 0, 0)),
            pl.BlockSpec((1, _L, _D), lambda h: (h, 0, 0)),
            pl.BlockSpec((1, _L, _D), lambda h: (h, 0, 0)),
            pl.BlockSpec((1, _L, _D), lambda h: (h, 0, 0)),
            pl.BlockSpec((1, _L, _D), lambda h: (h, 0, 0)),
        ],
        out_specs=pl.BlockSpec((1, _L, _D), lambda h: (h, 0, 0)),
        out_shape=jax.ShapeDtypeStruct((_H, _L, _D), jnp.float32),
    )(idx3, qh, kh, vh, ctx0)
    return ctx


def kernel(queries, keys, values, attn_mask):
    qh = jnp.transpose(queries[0], (1, 0, 2))
    kh = jnp.transpose(keys[0], (1, 0, 2))
    vh = jnp.transpose(values[0], (1, 0, 2))
    ctx = _run(qh, kh, vh)
    return jnp.transpose(ctx, (1, 0, 2))[None]
